# baseline (speedup 1.0000x reference)
import jax
import jax.numpy as jnp
from jax.experimental import pallas as pl
from jax.experimental.pallas import tpu as pltpu

_ROW_TILE = 256
_GROUP = 64
_STRIDE = _GROUP + 1
_LANES = 128
_N_ACC = 4


def _make_main_kernel(tm, chunks, v_total):
    def _main_kernel(i16_ref, cidx_ref, lneg_ref, table_ref,
                     out_ref, part_ref, t2, d2, ts_a, ts_b):
        @pl.when(pl.program_id(0) == 0)
        def _prep():
            def blk16(bb, _):
                for sub in range(16):
                    r0 = pl.multiple_of((bb * 16 + sub) * 8, 8)
                    x = table_ref[pl.ds(r0, 8), :]
                    m = jnp.max(x, axis=-1, keepdims=True)
                    s = jnp.sum(jnp.exp(x - m), axis=-1, keepdims=True)
                    d = (jnp.log(s) + m) - x
                    for v in range(chunks):
                        sl = pl.Slice(r0 * chunks + v, 8, chunks)
                        t2[sl, 0, :] = x[:, v * _LANES:(v + 1) * _LANES]
                        d2[sl, 0, :] = d[:, v * _LANES:(v + 1) * _LANES]
                return 0
            jax.lax.fori_loop(0, v_total // 128, blk16, 0)

        accs = [jnp.zeros((1, _LANES), jnp.float32) for _ in range(_N_ACC)]
        for g in range(tm // _GROUP):
            ts = ts_a if g % 2 == 0 else ts_b
            for mi in range(_GROUP):
                m = g * _GROUP + mi
                i16 = i16_ref[0, 0, m]
                slab = t2[pl.ds(i16, chunks), 0, :]
                ts[mi:mi + (chunks - 1) * _STRIDE + 1:_STRIDE, :] = slab
                chunk = d2[cidx_ref[0, 0, m]]
                accs[m % _N_ACC] = accs[m % _N_ACC] + pltpu.roll(
                    chunk, lneg_ref[0, 0, m], axis=1)
            r0 = g * _GROUP
            for j in range(chunks):
                out_ref[r0:r0 + _GROUP, j * _LANES:(j + 1) * _LANES] = (
                    ts[j * _STRIDE:j * _STRIDE + _GROUP, :])
        acc = (accs[0] + accs[1]) + (accs[2] + accs[3])
        part_ref[...] = acc.reshape(1, 1, _LANES)
    return _main_kernel


def kernel(idx, table, targets):
    B, T = idx.shape
    V = table.shape[0]
    N = B * T
    chunks = V // _LANES
    tm = min(_ROW_TILE, N)
    n_tiles = N // tm

    idx_f = idx.reshape(N).astype(jnp.int32)
    tgt_f = targets.reshape(N).astype(jnp.int32)
    i16 = (idx_f * chunks).reshape(n_tiles, 1, tm)
    cidx = (idx_f * chunks + (tgt_f // _LANES)).reshape(n_tiles, 1, tm)
    lneg = (-(tgt_f & (_LANES - 1))).reshape(n_tiles, 1, tm)

    smem_spec = pl.BlockSpec((1, 1, tm), lambda i: (i, 0, 0),
                             memory_space=pltpu.SMEM)
    scratch = pltpu.VMEM(((chunks - 1) * _STRIDE + _GROUP, _LANES),
                         jnp.float32)
    big = pltpu.VMEM((V * chunks, 1, _LANES), jnp.float32)
    logits, partials = pl.pallas_call(
        _make_main_kernel(tm, chunks, V),
        out_shape=(
            jax.ShapeDtypeStruct((N, V), jnp.float32),
            jax.ShapeDtypeStruct((n_tiles, 1, _LANES), jnp.float32),
        ),
        grid=(n_tiles,),
        in_specs=[
            smem_spec, smem_spec, smem_spec,
            pl.BlockSpec((V, V), lambda i: (0, 0)),
        ],
        out_specs=(
            pl.BlockSpec((tm, V), lambda i: (i, 0)),
            pl.BlockSpec((1, 1, _LANES), lambda i: (i, 0, 0)),
        ),
        scratch_shapes=[big, big, scratch, scratch],
        compiler_params=pltpu.CompilerParams(
            dimension_semantics=("arbitrary",),
            vmem_limit_bytes=58 * 1024 * 1024,
        ),
        cost_estimate=pl.CostEstimate(
            flops=2 * N * V,
            transcendentals=V * V,
            bytes_accessed=N * V * 4 + V * V * 4 + 3 * N * 4,
        ),
    )(i16, cidx, lneg, table)

    loss = jnp.sum(partials[:, 0, 0]) * (1.0 / N)
    return logits, loss


# speedup vs baseline: 1.2447x; 1.2447x over previous
import jax
import jax.numpy as jnp
from jax.experimental import pallas as pl
from jax.experimental.pallas import tpu as pltpu

_ROW_TILE = 256
_GROUP = 64
_STRIDE = _GROUP + 1
_LANES = 128
_N_ACC = 4


def _make_main_kernel(tm, chunks, v_total):
    def _main_kernel(i16_ref, c8_ref, sneg_ref, lneg_ref, table_ref,
                     out_ref, part_ref, t2, d2, ts_a, ts_b):
        @pl.when(pl.program_id(0) == 0)
        def _prep():
            def blk16(bb, _):
                for sub in range(16):
                    r0 = pl.multiple_of((bb * 16 + sub) * 8, 8)
                    x = table_ref[pl.ds(r0, 8), :]
                    m = jnp.max(x, axis=-1, keepdims=True)
                    s = jnp.sum(jnp.exp(x - m), axis=-1, keepdims=True)
                    d = (jnp.log(s) + m) - x
                    for v in range(chunks):
                        sl = pl.Slice(r0 * chunks + v, 8, chunks)
                        t2[sl, :] = x[:, v * _LANES:(v + 1) * _LANES]
                        d2[sl, :] = d[:, v * _LANES:(v + 1) * _LANES]
                return 0
            jax.lax.fori_loop(0, v_total // 128, blk16, 0)

        accs = [jnp.zeros((1, _LANES), jnp.float32) for _ in range(_N_ACC)]
        for g in range(tm // _GROUP):
            ts = ts_a if g % 2 == 0 else ts_b
            for mi in range(_GROUP):
                m = g * _GROUP + mi
                i16 = pl.multiple_of(i16_ref[0, 0, m],
                                     8 if chunks % 8 == 0 else chunks)
                slab = t2[pl.ds(i16, chunks), :]
                ts[mi:mi + (chunks - 1) * _STRIDE + 1:_STRIDE, :] = slab
                c8 = pl.multiple_of(c8_ref[0, 0, m], 8)
                blk = d2[pl.ds(c8, 8), :]
                blk = pltpu.roll(blk, sneg_ref[0, 0, m], axis=0)
                blk = pltpu.roll(blk, lneg_ref[0, 0, m], axis=1)
                accs[m % _N_ACC] = accs[m % _N_ACC] + blk[0:1, :]
            r0 = g * _GROUP
            for j in range(chunks):
                out_ref[r0:r0 + _GROUP, j * _LANES:(j + 1) * _LANES] = (
                    ts[j * _STRIDE:j * _STRIDE + _GROUP, :])
        acc = (accs[0] + accs[1]) + (accs[2] + accs[3])
        part_ref[...] = acc.reshape(1, 1, _LANES)
    return _main_kernel


def kernel(idx, table, targets):
    B, T = idx.shape
    V = table.shape[0]
    N = B * T
    chunks = V // _LANES
    tm = min(_ROW_TILE, N)
    n_tiles = N // tm

    idx_f = idx.reshape(N).astype(jnp.int32)
    tgt_f = targets.reshape(N).astype(jnp.int32)
    shape3 = (n_tiles, 1, tm)
    i16 = (idx_f * chunks).reshape(shape3)
    c8 = ((idx_f * chunks + tgt_f // _LANES) & ~jnp.int32(7)).reshape(shape3)
    sneg = (-((idx_f * chunks + tgt_f // _LANES) & 7)).reshape(shape3)
    lneg = (-(tgt_f & (_LANES - 1))).reshape(shape3)

    smem_spec = pl.BlockSpec((1, 1, tm), lambda i: (i, 0, 0),
                             memory_space=pltpu.SMEM)
    scratch = pltpu.VMEM(((chunks - 1) * _STRIDE + _GROUP, _LANES),
                         jnp.float32)
    big = pltpu.VMEM((V * chunks, _LANES), jnp.float32)
    logits, partials = pl.pallas_call(
        _make_main_kernel(tm, chunks, V),
        out_shape=(
            jax.ShapeDtypeStruct((N, V), jnp.float32),
            jax.ShapeDtypeStruct((n_tiles, 1, _LANES), jnp.float32),
        ),
        grid=(n_tiles,),
        in_specs=[
            smem_spec, smem_spec, smem_spec, smem_spec,
            pl.BlockSpec((V, V), lambda i: (0, 0)),
        ],
        out_specs=(
            pl.BlockSpec((tm, V), lambda i: (i, 0)),
            pl.BlockSpec((1, 1, _LANES), lambda i: (i, 0, 0)),
        ),
        scratch_shapes=[big, big, scratch, scratch],
        compiler_params=pltpu.CompilerParams(
            dimension_semantics=("arbitrary",),
            vmem_limit_bytes=58 * 1024 * 1024,
        ),
        cost_estimate=pl.CostEstimate(
            flops=2 * N * V,
            transcendentals=V * V,
            bytes_accessed=N * V * 4 + V * V * 4 + 4 * N * 4,
        ),
    )(i16, c8, sneg, lneg, table)

    loss = jnp.sum(partials[:, 0, 0]) * (1.0 / N)
    return logits, loss
